# baseline (speedup 1.0000x reference)
import functools

import jax
import jax.numpy as jnp
from jax import lax
from jax.experimental import pallas as pl
from jax.experimental.pallas import tpu as pltpu
from jax.experimental.pallas import tpu_sc as plsc

_NC = 2
_NS = 16
_CHUNK = 32
_NBUF = 4


def _ring_body(idx_v, gathers, writes, per_w, n_grp):
    n_ch = n_grp * _NBUF
    for j in range(_NBUF - 1):
        for gth in gathers(j, j):
            gth.start()

    def body(g, _):
        for b in range(_NBUF):
            j = g * _NBUF + b
            for gth in gathers(j, b):
                gth.wait()
            for w in writes(j, b):
                w.start()
            bp = (b - 1) % _NBUF

            def drain_prev():
                for w in writes(j - 1, bp):
                    w.wait()

            def refill():
                drain_prev()
                for gth in gathers(j + _NBUF - 1, bp):
                    gth.start()

            if b == 0:
                pl.when(g >= 1)(drain_prev)
                for gth in gathers(j + _NBUF - 1, bp):
                    gth.start()
            else:
                pl.when(g < n_grp - 1)(refill)
        return ()

    lax.fori_loop(0, n_grp, body, (), unroll=False)
    for j in range(n_ch - _NBUF, n_ch):
        for w in writes(j, j % _NBUF):
            w.wait()


@functools.lru_cache(maxsize=None)
def _make_col_gather(M, D, n_loc, width, tc_tiling):
    NW = _NC * _NS
    per_w = n_loc // NW
    n_ch = per_w // _CHUNK
    assert n_ch % _NBUF == 0 and n_ch >= 2 * _NBUF
    n_grp = n_ch // _NBUF
    mesh = plsc.VectorSubcoreMesh(core_axis_name="c", subcore_axis_name="s")

    @functools.partial(
        pl.kernel,
        mesh=mesh,
        out_type=jax.ShapeDtypeStruct((n_loc, width), jnp.float32),
        scratch_types=[
            pltpu.VMEM((per_w,), jnp.int32),
            [pltpu.VMEM((_CHUNK, width), jnp.float32) for _ in range(_NBUF)],
            [pltpu.SemaphoreType.DMA for _ in range(_NBUF)],
            [pltpu.SemaphoreType.DMA for _ in range(_NBUF)],
        ],
        compiler_params=pltpu.CompilerParams(use_tc_tiling_on_sc=tc_tiling),
    )
    def gather_kernel(src_hbm, loc_hbm, out_hbm, idx_v, bufs, gsems, wsems):
        wid = lax.axis_index("s") * _NC + lax.axis_index("c")
        base = wid * per_w

        def gathers(j, b):
            idx = idx_v.at[pl.ds(j * _CHUNK, _CHUNK)]
            if width == D:
                src = src_hbm.at[idx]
            else:
                src = src_hbm.at[idx, pl.ds(0, width)]
            return (pltpu.make_async_copy(src, bufs[b], gsems[b]),)

        def writes(j, b):
            row0 = base + j * _CHUNK
            return (
                pltpu.make_async_copy(
                    bufs[b],
                    out_hbm.at[pl.ds(row0, _CHUNK), pl.ds(0, width)],
                    wsems[b]),
            )

        pltpu.sync_copy(loc_hbm.at[pl.ds(base, per_w)], idx_v)
        _ring_body(idx_v, gathers, writes, per_w, n_grp)

    return gather_kernel


@functools.lru_cache(maxsize=None)
def _make_col_slice(M, D, col0, width):
    NW = _NC * _NS
    rows_w = M // NW
    RC = 512
    n_ch = rows_w // RC
    mesh = plsc.VectorSubcoreMesh(core_axis_name="c", subcore_axis_name="s")

    @functools.partial(
        pl.kernel,
        mesh=mesh,
        out_type=jax.ShapeDtypeStruct((M, width), jnp.float32),
        scratch_types=[
            [pltpu.VMEM((RC, width), jnp.float32) for _ in range(2)],
            [pltpu.SemaphoreType.DMA for _ in range(2)],
            [pltpu.SemaphoreType.DMA for _ in range(2)],
        ],
    )
    def slice_kernel(src_hbm, out_hbm, bufs, rsems, wsems):
        wid = lax.axis_index("s") * _NC + lax.axis_index("c")
        base = wid * rows_w

        def read(j, b):
            return pltpu.make_async_copy(
                src_hbm.at[pl.ds(base + j * RC, RC), pl.ds(col0, width)],
                bufs[b], rsems[b])

        def write(j, b):
            return pltpu.make_async_copy(
                bufs[b], out_hbm.at[pl.ds(base + j * RC, RC), pl.ds(0, width)],
                wsems[b])

        read(0, 0).start()

        def pair(g, _):
            for b in range(2):
                j = g * 2 + b
                read(j, b).wait()
                write(j, b).start()

                def refill():
                    write(j - 1, 1 - b).wait()
                    read(j + 1, 1 - b).start()

                if b == 0:
                    pl.when(g >= 1)(lambda: write(j - 1, 1).wait())
                    read(j + 1, 1).start()
                else:
                    pl.when(g < n_ch // 2 - 1)(refill)
            return ()

        lax.fori_loop(0, n_ch // 2, pair, (), unroll=False)
        for j in (n_ch - 2, n_ch - 1):
            write(j, j % 2).wait()

    return slice_kernel


def kernel(kv_buffer, loc, cache_k_nope, cache_k_rope):
    M, D = kv_buffer.shape
    n_loc = loc.shape[0]
    nope_dim = cache_k_nope.shape[-1]
    rope_dim = cache_k_rope.shape[-1]
    rope_src = _make_col_slice(M, D, nope_dim, rope_dim)(kv_buffer)
    nope = _make_col_gather(M, D, n_loc, nope_dim, True)(kv_buffer, loc)
    rope = _make_col_gather(M, rope_dim, n_loc, rope_dim, False)(rope_src, loc)
    return nope, rope


# speedup vs baseline: 1.0589x; 1.0589x over previous
import functools

import jax
import jax.numpy as jnp
from jax import lax
from jax.experimental import pallas as pl
from jax.experimental.pallas import tpu as pltpu
from jax.experimental.pallas import tpu_sc as plsc

_NC = 2
_NS = 16
_CHUNK = 32
_NBUF = 4
_LANE = 128


def _ring_body(gathers, writes, n_grp):
    n_ch = n_grp * _NBUF
    for j in range(_NBUF - 1):
        for gth in gathers(j, j):
            gth.start()

    def body(g, _):
        for b in range(_NBUF):
            j = g * _NBUF + b
            for gth in gathers(j, b):
                gth.wait()
            yield j, b
            bp = (b - 1) % _NBUF

            def drain_prev():
                for w in writes(j - 1, bp):
                    w.wait()

            def refill():
                drain_prev()
                for gth in gathers(j + _NBUF - 1, bp):
                    gth.start()

            if b == 0:
                pl.when(g >= 1)(drain_prev)
                for gth in gathers(j + _NBUF - 1, bp):
                    gth.start()
            else:
                pl.when(g < n_grp - 1)(refill)

    return body


def _run_ring(gathers, writes, n_grp, stage=None):
    n_ch = n_grp * _NBUF
    gen_body = _ring_body(gathers, writes, n_grp)

    def body(g, _):
        for j, b in gen_body(g, None):
            if stage is not None:
                stage(j, b)
            for w in writes(j, b):
                w.start()
        return ()

    lax.fori_loop(0, n_grp, body, (), unroll=False)
    for j in range(n_ch - _NBUF, n_ch):
        for w in writes(j, j % _NBUF):
            w.wait()


@functools.lru_cache(maxsize=None)
def _make_nope_gather(M, D, n_loc, width):
    NW = _NC * _NS
    per_w = n_loc // NW
    n_grp = per_w // _CHUNK // _NBUF
    mesh = plsc.VectorSubcoreMesh(core_axis_name="c", subcore_axis_name="s")

    @functools.partial(
        pl.kernel,
        mesh=mesh,
        out_type=jax.ShapeDtypeStruct((n_loc, width), jnp.float32),
        scratch_types=[
            pltpu.VMEM((per_w,), jnp.int32),
            [pltpu.VMEM((_CHUNK, width), jnp.float32) for _ in range(_NBUF)],
            [pltpu.SemaphoreType.DMA for _ in range(_NBUF)],
            [pltpu.SemaphoreType.DMA for _ in range(_NBUF)],
        ],
    )
    def nope_kernel(src_hbm, loc_hbm, out_hbm, idx_v, bufs, gsems, wsems):
        wid = lax.axis_index("s") * _NC + lax.axis_index("c")
        base = wid * per_w

        def gathers(j, b):
            idx = idx_v.at[pl.ds(j * _CHUNK, _CHUNK)]
            return (pltpu.make_async_copy(
                src_hbm.at[idx, pl.ds(0, width)], bufs[b], gsems[b]),)

        def writes(j, b):
            row0 = base + j * _CHUNK
            return (pltpu.make_async_copy(
                bufs[b], out_hbm.at[pl.ds(row0, _CHUNK), pl.ds(0, width)],
                wsems[b]),)

        pltpu.sync_copy(loc_hbm.at[pl.ds(base, per_w)], idx_v)
        _run_ring(gathers, writes, n_grp)

    return nope_kernel


@functools.lru_cache(maxsize=None)
def _make_pairs_slice(M, D, col0, width):
    NW = _NC * _NS
    rows_w = M // NW
    RC = 256
    n_ch = rows_w // RC
    flat_c = RC * width
    mesh = plsc.VectorSubcoreMesh(core_axis_name="c", subcore_axis_name="s")

    @functools.partial(
        pl.kernel,
        mesh=mesh,
        out_type=jax.ShapeDtypeStruct((M * width,), jnp.float32),
        scratch_types=[
            [pltpu.VMEM((RC, width), jnp.float32) for _ in range(2)],
            [pltpu.VMEM((flat_c,), jnp.float32) for _ in range(2)],
            [pltpu.SemaphoreType.DMA for _ in range(2)],
            [pltpu.SemaphoreType.DMA for _ in range(2)],
        ],
    )
    def slice_kernel(src_hbm, out_hbm, bufs, fbufs, rsems, wsems):
        wid = lax.axis_index("s") * _NC + lax.axis_index("c")
        r0 = wid * rows_w

        def read(j, b):
            return pltpu.make_async_copy(
                src_hbm.at[pl.ds(r0 + j * RC, RC), pl.ds(col0, width)],
                bufs[b], rsems[b])

        def write(j, b):
            return pltpu.make_async_copy(
                fbufs[b], out_hbm.at[pl.ds((r0 + j * RC) * width, flat_c)],
                wsems[b])

        def flatten(b):
            def cp(r, _):
                for c in range(width // 16):
                    fbufs[b][pl.ds(r * width + c * 16, 16)] = (
                        bufs[b][r, pl.ds(c * 16, 16)])
                return ()
            lax.fori_loop(0, RC, cp, (), unroll=False)

        read(0, 0).start()

        def pair(g, _):
            for b in range(2):
                j = g * 2 + b
                read(j, b).wait()
                if b == 0:
                    pl.when(g >= 1)(lambda: write(2 * g - 1, 1).wait())
                    read(j + 1, 1).start()
                else:
                    def refill():
                        write(j - 1, 0).wait()
                        read(j + 1, 0).start()
                    pl.when(g < n_ch // 2 - 1)(refill)
                flatten(b)
                write(j, b).start()
            return ()

        lax.fori_loop(0, n_ch // 2, pair, (), unroll=False)
        for j in (n_ch - 2, n_ch - 1):
            write(j, j % 2).wait()

    return slice_kernel


@functools.lru_cache(maxsize=None)
def _make_rope_gather(M, n_loc, rope_dim):
    NW = _NC * _NS
    per_w = n_loc // NW
    n_grp = per_w // _CHUNK // _NBUF
    mesh = plsc.VectorSubcoreMesh(core_axis_name="c", subcore_axis_name="s")

    @functools.partial(
        pl.kernel,
        mesh=mesh,
        out_type=jax.ShapeDtypeStruct((n_loc, rope_dim), jnp.float32),
        scratch_types=[
            pltpu.VMEM((per_w,), jnp.int32),
            pltpu.VMEM((per_w,), jnp.int32),
            [pltpu.VMEM((_CHUNK, _LANE), jnp.float32) for _ in range(_NBUF)],
            [pltpu.VMEM((_CHUNK, rope_dim), jnp.float32) for _ in range(_NBUF)],
            [pltpu.SemaphoreType.DMA for _ in range(_NBUF)],
            [pltpu.SemaphoreType.DMA for _ in range(_NBUF)],
        ],
    )
    def rope_kernel(pairs_hbm, loc_hbm, out_hbm,
                    idx_v, idx2_v, rbufs, sbufs, gsems, wsems):
        wid = lax.axis_index("s") * _NC + lax.axis_index("c")
        base = wid * per_w

        def gathers(j, b):
            idx = idx2_v.at[pl.ds(j * _CHUNK, _CHUNK)]
            return (pltpu.make_async_copy(pairs_hbm.at[idx], rbufs[b], gsems[b]),)

        def writes(j, b):
            row0 = base + j * _CHUNK
            return (pltpu.make_async_copy(
                sbufs[b], out_hbm.at[pl.ds(row0, _CHUNK), pl.ds(0, rope_dim)],
                wsems[b]),)

        def stage(j, b):
            for half in range(_CHUNK // 16):
                hv = (idx_v[pl.ds(j * _CHUNK + half * 16, 16)] & 1) * rope_dim
                for r in range(16):
                    rr = half * 16 + r
                    h = hv[r]
                    for c in range(rope_dim // 16):
                        sbufs[b][rr, pl.ds(c * 16, 16)] = (
                            rbufs[b][rr, pl.ds(h + c * 16, 16)])

        pltpu.sync_copy(loc_hbm.at[pl.ds(base, per_w)], idx_v)

        def prep(i, _):
            idx2_v[pl.ds(i * 16, 16)] = idx_v[pl.ds(i * 16, 16)] >> 1
            return ()

        lax.fori_loop(0, per_w // 16, prep, (), unroll=False)
        _run_ring(gathers, writes, n_grp, stage=stage)

    return rope_kernel


def kernel(kv_buffer, loc, cache_k_nope, cache_k_rope):
    M, D = kv_buffer.shape
    n_loc = loc.shape[0]
    nope_dim = cache_k_nope.shape[-1]
    rope_dim = cache_k_rope.shape[-1]
    pairs_flat = _make_pairs_slice(M, D, nope_dim, rope_dim)(kv_buffer)
    pairs = jnp.reshape(pairs_flat, (M * rope_dim // _LANE, _LANE))
    nope = _make_nope_gather(M, D, n_loc, nope_dim)(kv_buffer, loc)
    rope = _make_rope_gather(M, n_loc, rope_dim)(pairs, loc)
    return nope, rope
